# baseline (speedup 1.0000x reference)
import functools

import jax
import jax.numpy as jnp
from jax import lax
from jax.experimental import pallas as pl
from jax.experimental.pallas import tpu as pltpu
from jax.experimental.pallas import tpu_sc as plsc

NC = 2
NS = 16
NW = NC * NS
LANES = 16

NNODES = 100000
ROWS = 784
NPAD = ROWS * 128
SLICE = NPAD // NS

KB = 128
NEDGES = 3200000
G = 8
NPG = 98
TROWS = NPG * G
EROWS = NW * TROWS
EPAD = EROWS * KB
RING = 2
NI = NPG // RING

_mesh = plsc.VectorSubcoreMesh(
    core_axis_name="c", subcore_axis_name="s", num_cores=NC, num_subcores=NS)


def _zero_slice(stage, acc, sid):
    def zstep(i, carry):
        stage[pl.ds(i * LANES, LANES)] = jnp.zeros((LANES,), jnp.float32)
        return carry
    lax.fori_loop(0, SLICE // LANES, zstep, 0)
    pltpu.sync_copy(stage, acc.at[pl.ds(sid * SLICE, SLICE)])


def _writeout(stage, acc, out_hbm, cid, sid):
    off = sid * SLICE
    pltpu.sync_copy(acc.at[pl.ds(off, SLICE)], stage)
    pltpu.sync_copy(stage, out_hbm.at[pl.ds(cid * NPAD + off, SLICE)])


@functools.partial(
    pl.kernel,
    mesh=_mesh,
    out_type=jax.ShapeDtypeStruct((NC * NPAD,), jnp.float32),
    scratch_types=[
        [pltpu.VMEM((G, KB), jnp.int32) for _ in range(RING)],
        pltpu.VMEM((G, KB), jnp.float32),
        pltpu.VMEM((SLICE,), jnp.float32),
        pltpu.VMEM_SHARED((NPAD,), jnp.float32),
        [pltpu.SemaphoreType.DMA for _ in range(RING)],
        [pltpu.SemaphoreType.DMA for _ in range(RING)],
    ],
)
def _deg_pass(dst_hbm, out_hbm, idxd, ones, stage, acc, semL, semS):
    cid = lax.axis_index("c")
    sid = lax.axis_index("s")
    wid = sid * NC + cid

    for j in range(G):
        for c in range(KB // LANES):
            ones[j, pl.ds(c * LANES, LANES)] = jnp.ones((LANES,), jnp.float32)

    _zero_slice(stage, acc, sid)
    plsc.subcore_barrier()

    row0 = wid * TROWS

    def fire_load(b, g):
        pltpu.async_copy(dst_hbm.at[pl.ds(row0 + g * G, G)], idxd[b], semL[b])

    for b in range(RING):
        fire_load(b, b)

    def step(i, carry):
        for b in range(RING):
            pltpu.make_async_copy(
                dst_hbm.at[pl.ds(row0, G)], idxd[b], semL[b]).wait()
            for j in range(G):
                pltpu.async_copy(
                    ones.at[j], acc.at[idxd[b].at[j]], semS[b], add=True)
        for b in range(RING):
            for j in range(G):
                pltpu.make_async_copy(
                    ones.at[j], acc.at[idxd[b].at[j]], semS[b]).wait()

            @pl.when(i + 1 < NI)
            def _():
                fire_load(b, (i + 1) * RING + b)
        return carry
    lax.fori_loop(0, NI, step, 0)

    plsc.subcore_barrier()
    _writeout(stage, acc, out_hbm, cid, sid)


@functools.partial(
    pl.kernel,
    mesh=_mesh,
    out_type=jax.ShapeDtypeStruct((NC * NPAD,), jnp.float32),
    scratch_types=[
        [pltpu.VMEM((G, KB), jnp.int32) for _ in range(RING)],
        [pltpu.VMEM((G, KB), jnp.int32) for _ in range(RING)],
        [pltpu.VMEM((G, KB), jnp.float32) for _ in range(RING)],
        pltpu.VMEM((SLICE,), jnp.float32),
        pltpu.VMEM_SHARED((NPAD,), jnp.float32),
        [pltpu.SemaphoreType.DMA for _ in range(RING)],
        [pltpu.SemaphoreType.DMA for _ in range(RING)],
        [pltpu.SemaphoreType.DMA for _ in range(RING)],
    ],
)
def _seg_pass(src_hbm, dst_hbm, tab_hbm, out_hbm,
              idxs, idxd, vals, stage, acc, semL, semG, semS):
    cid = lax.axis_index("c")
    sid = lax.axis_index("s")
    wid = sid * NC + cid

    _zero_slice(stage, acc, sid)
    plsc.subcore_barrier()

    row0 = wid * TROWS

    def fire_loads(b, g):
        r = row0 + g * G
        pltpu.async_copy(src_hbm.at[pl.ds(r, G)], idxs[b], semL[b])
        pltpu.async_copy(dst_hbm.at[pl.ds(r, G)], idxd[b], semL[b])

    for b in range(RING):
        fire_loads(b, b)

    def step(i, carry):
        for b in range(RING):
            pltpu.make_async_copy(
                src_hbm.at[pl.ds(row0, G)], idxs[b], semL[b]).wait()
            pltpu.make_async_copy(
                dst_hbm.at[pl.ds(row0, G)], idxd[b], semL[b]).wait()
            for j in range(G):
                pltpu.async_copy(
                    tab_hbm.at[idxs[b].at[j]], vals[b].at[j], semG[b])
        for b in range(RING):
            for j in range(G):
                pltpu.make_async_copy(
                    tab_hbm.at[idxs[b].at[j]], vals[b].at[j], semG[b]).wait()
            for j in range(G):
                pltpu.async_copy(
                    vals[b].at[j], acc.at[idxd[b].at[j]], semS[b], add=True)
        for b in range(RING):
            for j in range(G):
                pltpu.make_async_copy(
                    vals[b].at[j], acc.at[idxd[b].at[j]], semS[b]).wait()

            @pl.when(i + 1 < NI)
            def _():
                fire_loads(b, (i + 1) * RING + b)
        return carry
    lax.fori_loop(0, NI, step, 0)

    plsc.subcore_barrier()
    _writeout(stage, acc, out_hbm, cid, sid)


def _node1_body(cnt_ref, x_ref, dis_ref, p_ref):
    deg = cnt_ref[0] + cnt_ref[1] + 1.0
    dis = lax.rsqrt(deg)
    dis_ref[...] = dis
    p_ref[...] = x_ref[...] * dis


def _node2_body(s1_ref, p_ref, dis_ref, w1_ref, b1_ref, w2_ref, q_ref):
    dis = dis_ref[...]
    s1 = dis * (s1_ref[0] + s1_ref[1] + p_ref[...])
    acc = jnp.zeros_like(s1)
    for k in range(16):
        h = jnp.maximum(s1 * w1_ref[0, k] + b1_ref[0, k], 0.0)
        acc = acc + h * w2_ref[0, k]
    q_ref[...] = acc * dis


def _node3_body(s2_ref, q_ref, dis_ref, b2_ref, out_ref):
    out_ref[...] = dis_ref[...] * (s2_ref[0] + s2_ref[1] + q_ref[...]) + b2_ref[0, 0]


_f32 = jnp.float32
_mat = jax.ShapeDtypeStruct((ROWS, 128), _f32)
_smem_spec = pl.BlockSpec(memory_space=pltpu.SMEM)

_node1 = pl.pallas_call(_node1_body, out_shape=(_mat, _mat))
_node2 = pl.pallas_call(
    _node2_body,
    in_specs=[pl.BlockSpec((2, ROWS, 128)), pl.BlockSpec((ROWS, 128)),
              pl.BlockSpec((ROWS, 128)), _smem_spec, _smem_spec, _smem_spec],
    out_shape=_mat)
_node3 = pl.pallas_call(
    _node3_body,
    in_specs=[pl.BlockSpec((2, ROWS, 128)), pl.BlockSpec((ROWS, 128)),
              pl.BlockSpec((ROWS, 128)), _smem_spec],
    out_shape=_mat)


def kernel(x, edge_index, W1, b1, W2, b2):
    n = x.shape[0]
    e = edge_index.shape[1]
    src2 = jnp.concatenate(
        [edge_index[0], jnp.zeros((EPAD - e,), jnp.int32)]).reshape(EROWS, KB)
    dst2 = jnp.concatenate(
        [edge_index[1], jnp.full((EPAD - e,), n, jnp.int32)]).reshape(EROWS, KB)

    cnt = _deg_pass(dst2).reshape(NC, ROWS, 128)
    xpad = jnp.concatenate(
        [x[:, 0], jnp.zeros((NPAD - n,), _f32)]).reshape(ROWS, 128)
    dis, p = _node1(cnt, xpad)

    s1 = _seg_pass(src2, dst2, p.reshape(NPAD)).reshape(NC, ROWS, 128)
    q = _node2(s1, p, dis, W1.reshape(1, 16), b1.reshape(1, 16),
               W2.reshape(1, 16))

    s2 = _seg_pass(src2, dst2, q.reshape(NPAD)).reshape(NC, ROWS, 128)
    out = _node3(s2, q, dis, b2.reshape(1, 1))
    return out.reshape(NPAD)[:n].reshape(n, 1)


# speedup vs baseline: 1.5816x; 1.5816x over previous
import functools

import jax
import jax.numpy as jnp
from jax import lax
from jax.experimental import pallas as pl
from jax.experimental.pallas import tpu as pltpu
from jax.experimental.pallas import tpu_sc as plsc

NC = 2
NS = 16
NW = NC * NS
LANES = 16

NNODES = 100000
ROWS = 784
NPAD = ROWS * 128
SLICE = NPAD // NS

KB = 128
NEDGES = 3200000
G = 8
EROWS = NEDGES // KB
NGRP = EROWS // G
GQ = NGRP // NW
GR = NGRP - GQ * NW
RING = 2

_mesh = plsc.VectorSubcoreMesh(
    core_axis_name="c", subcore_axis_name="s", num_cores=NC, num_subcores=NS)


def _zero_slice(stage, acc, sid):
    def zstep(i, carry):
        stage[pl.ds(i * LANES, LANES)] = jnp.zeros((LANES,), jnp.float32)
        return carry
    lax.fori_loop(0, SLICE // LANES, zstep, 0)
    pltpu.sync_copy(stage, acc.at[pl.ds(sid * SLICE, SLICE)])


def _writeout(stage, acc, out_hbm, cid, sid):
    off = sid * SLICE
    pltpu.sync_copy(acc.at[pl.ds(off, SLICE)], stage)
    pltpu.sync_copy(stage, out_hbm.at[pl.ds(cid * NPAD + off, SLICE)])


def _shard(wid):
    g0 = wid * GQ + jnp.minimum(wid, GR)
    ng = GQ + jnp.where(wid < GR, 1, 0)
    return g0, ng


@functools.partial(
    pl.kernel,
    mesh=_mesh,
    out_type=jax.ShapeDtypeStruct((NC * NPAD,), jnp.float32),
    scratch_types=[
        [pltpu.VMEM((G, KB), jnp.int32) for _ in range(RING)],
        pltpu.VMEM((G, KB), jnp.float32),
        pltpu.VMEM((SLICE,), jnp.float32),
        pltpu.VMEM_SHARED((NPAD,), jnp.float32),
        [pltpu.SemaphoreType.DMA for _ in range(RING)],
        [pltpu.SemaphoreType.DMA for _ in range(RING)],
    ],
)
def _deg_pass(dst_hbm, out_hbm, idxd, ones, stage, acc, semL, semS):
    cid = lax.axis_index("c")
    sid = lax.axis_index("s")
    wid = sid * NC + cid

    for j in range(G):
        for c in range(KB // LANES):
            ones[j, pl.ds(c * LANES, LANES)] = jnp.ones((LANES,), jnp.float32)

    _zero_slice(stage, acc, sid)
    plsc.subcore_barrier()

    g0, ng = _shard(wid)

    def fire_load(b, g):
        pltpu.async_copy(dst_hbm.at[pl.ds((g0 + g) * G, G)], idxd[b], semL[b])

    def wait_load(b):
        pltpu.make_async_copy(dst_hbm.at[pl.ds(0, G)], idxd[b], semL[b]).wait()

    def scat(b):
        for j in range(G):
            pltpu.async_copy(
                ones.at[j], acc.at[idxd[b].at[j]], semS[b], add=True)

    def wait_scat(b):
        for j in range(G):
            pltpu.make_async_copy(
                ones.at[j], acc.at[idxd[b].at[j]], semS[b]).wait()

    for b in range(RING):
        fire_load(b, b)

    def step(i, carry):
        for b in range(RING):
            wait_load(b)
            scat(b)
        for b in range(RING):
            wait_scat(b)
            gnext = (i + 1) * RING + b

            @pl.when(gnext < ng)
            def _():
                fire_load(b, gnext)
        return carry
    lax.fori_loop(0, ng // RING, step, 0)

    @pl.when(ng % RING == 1)
    def _():
        wait_load(0)
        scat(0)
        wait_scat(0)

    plsc.subcore_barrier()
    _writeout(stage, acc, out_hbm, cid, sid)


@functools.partial(
    pl.kernel,
    mesh=_mesh,
    out_type=jax.ShapeDtypeStruct((NC * NPAD,), jnp.float32),
    scratch_types=[
        [pltpu.VMEM((G, KB), jnp.int32) for _ in range(RING)],
        [pltpu.VMEM((G, KB), jnp.int32) for _ in range(RING)],
        [pltpu.VMEM((G, KB), jnp.float32) for _ in range(RING)],
        pltpu.VMEM((SLICE,), jnp.float32),
        pltpu.VMEM_SHARED((NPAD,), jnp.float32),
        pltpu.VMEM_SHARED((NPAD,), jnp.float32),
        [pltpu.SemaphoreType.DMA for _ in range(RING)],
        [pltpu.SemaphoreType.DMA for _ in range(RING)],
        [pltpu.SemaphoreType.DMA for _ in range(RING)],
    ],
)
def _seg_pass(src_hbm, dst_hbm, tab_hbm, out_hbm,
              idxs, idxd, vals, stage, tab, acc, semL, semG, semS):
    cid = lax.axis_index("c")
    sid = lax.axis_index("s")
    wid = sid * NC + cid

    toff = sid * SLICE
    pltpu.sync_copy(tab_hbm.at[pl.ds(toff, SLICE)], stage)
    pltpu.sync_copy(stage, tab.at[pl.ds(toff, SLICE)])
    _zero_slice(stage, acc, sid)
    plsc.subcore_barrier()

    g0, ng = _shard(wid)

    def fire_loads(b, g):
        r = (g0 + g) * G
        pltpu.async_copy(src_hbm.at[pl.ds(r, G)], idxs[b], semL[b])
        pltpu.async_copy(dst_hbm.at[pl.ds(r, G)], idxd[b], semL[b])

    def wait_loads(b):
        pltpu.make_async_copy(src_hbm.at[pl.ds(0, G)], idxs[b], semL[b]).wait()
        pltpu.make_async_copy(dst_hbm.at[pl.ds(0, G)], idxd[b], semL[b]).wait()

    def gath(b):
        for j in range(G):
            pltpu.async_copy(
                tab.at[idxs[b].at[j]], vals[b].at[j], semG[b])

    def wait_gath(b):
        for j in range(G):
            pltpu.make_async_copy(
                tab.at[idxs[b].at[j]], vals[b].at[j], semG[b]).wait()

    def scat(b):
        for j in range(G):
            pltpu.async_copy(
                vals[b].at[j], acc.at[idxd[b].at[j]], semS[b], add=True)

    def wait_scat(b):
        for j in range(G):
            pltpu.make_async_copy(
                vals[b].at[j], acc.at[idxd[b].at[j]], semS[b]).wait()

    for b in range(RING):
        fire_loads(b, b)

    def step(i, carry):
        for b in range(RING):
            wait_loads(b)
            gath(b)
        for b in range(RING):
            wait_gath(b)
            scat(b)
        for b in range(RING):
            wait_scat(b)
            gnext = (i + 1) * RING + b

            @pl.when(gnext < ng)
            def _():
                fire_loads(b, gnext)
        return carry
    lax.fori_loop(0, ng // RING, step, 0)

    @pl.when(ng % RING == 1)
    def _():
        wait_loads(0)
        gath(0)
        wait_gath(0)
        scat(0)
        wait_scat(0)

    plsc.subcore_barrier()
    _writeout(stage, acc, out_hbm, cid, sid)


def _node1_body(cnt_ref, x_ref, dis_ref, p_ref):
    deg = cnt_ref[0] + cnt_ref[1] + 1.0
    dis = lax.rsqrt(deg)
    dis_ref[...] = dis
    p_ref[...] = x_ref[...] * dis


def _node2_body(s1_ref, p_ref, dis_ref, w1_ref, b1_ref, w2_ref, q_ref):
    dis = dis_ref[...]
    s1 = dis * (s1_ref[0] + s1_ref[1] + p_ref[...])
    acc = jnp.zeros_like(s1)
    for k in range(16):
        h = jnp.maximum(s1 * w1_ref[0, k] + b1_ref[0, k], 0.0)
        acc = acc + h * w2_ref[0, k]
    q_ref[...] = acc * dis


def _node3_body(s2_ref, q_ref, dis_ref, b2_ref, out_ref):
    out_ref[...] = dis_ref[...] * (s2_ref[0] + s2_ref[1] + q_ref[...]) + b2_ref[0, 0]


_f32 = jnp.float32
_mat = jax.ShapeDtypeStruct((ROWS, 128), _f32)
_smem_spec = pl.BlockSpec(memory_space=pltpu.SMEM)

_node1 = pl.pallas_call(_node1_body, out_shape=(_mat, _mat))
_node2 = pl.pallas_call(
    _node2_body,
    in_specs=[pl.BlockSpec((2, ROWS, 128)), pl.BlockSpec((ROWS, 128)),
              pl.BlockSpec((ROWS, 128)), _smem_spec, _smem_spec, _smem_spec],
    out_shape=_mat)
_node3 = pl.pallas_call(
    _node3_body,
    in_specs=[pl.BlockSpec((2, ROWS, 128)), pl.BlockSpec((ROWS, 128)),
              pl.BlockSpec((ROWS, 128)), _smem_spec],
    out_shape=_mat)


def kernel(x, edge_index, W1, b1, W2, b2):
    n = x.shape[0]
    src2 = edge_index[0].reshape(EROWS, KB)
    dst2 = edge_index[1].reshape(EROWS, KB)

    cnt = _deg_pass(dst2).reshape(NC, ROWS, 128)
    xpad = jnp.concatenate(
        [x[:, 0], jnp.zeros((NPAD - n,), _f32)]).reshape(ROWS, 128)
    dis, p = _node1(cnt, xpad)

    s1 = _seg_pass(src2, dst2, p.reshape(NPAD)).reshape(NC, ROWS, 128)
    q = _node2(s1, p, dis, W1.reshape(1, 16), b1.reshape(1, 16),
               W2.reshape(1, 16))

    s2 = _seg_pass(src2, dst2, q.reshape(NPAD)).reshape(NC, ROWS, 128)
    out = _node3(s2, q, dis, b2.reshape(1, 1))
    return out.reshape(NPAD)[:n].reshape(n, 1)
